# baseline (speedup 1.0000x reference)
import functools

import jax
import jax.numpy as jnp
from jax import lax
from jax.experimental import pallas as pl
from jax.experimental.pallas import tpu as pltpu
from jax.experimental.pallas import tpu_sc as plsc

_BATCH = 16384
_D = 64
_NC = 2
_NS = 16
_NW = _NC * _NS
_BPW = _BATCH // _NW
_CHUNK = 128
_NCHUNK = _BPW // _CHUNK
_L = 16


def _sc_body(users_h, pos_h, neg_h, ut_h, it_h,
             xp_out, reg_out,
             idx_u, idx_p, idx_n, ru, rp, rn, parts, racc, sem):
    wid = lax.axis_index("s") * _NC + lax.axis_index("c")
    base = wid * _BPW

    for j in range(_NCHUNK):
        sl = pl.ds(base + j * _CHUNK, _CHUNK)
        pltpu.sync_copy(users_h.at[sl], idx_u.at[j])
        pltpu.sync_copy(pos_h.at[sl], idx_p.at[j])
        pltpu.sync_copy(neg_h.at[sl], idx_n.at[j])

    copies = []
    for j in range(_NCHUNK):
        dst = pl.ds(j * _CHUNK, _CHUNK)
        copies.append(pltpu.async_copy(ut_h.at[idx_u.at[j]], ru.at[dst], sem))
        copies.append(pltpu.async_copy(it_h.at[idx_p.at[j]], rp.at[dst], sem))
        copies.append(pltpu.async_copy(it_h.at[idx_n.at[j]], rn.at[dst], sem))
    for c in copies:
        c.wait()

    zero = jnp.zeros((_L,), jnp.float32)

    def row_body(i, sacc):
        pv = zero
        for kk in range(_D // _L):
            sl = pl.ds(kk * _L, _L)
            u = ru[i, sl]
            p = rp[i, sl]
            n = rn[i, sl]
            pv = pv + u * (p - n)
            sacc = sacc + u * u + p * p + n * n
        parts[i, :] = pv
        return sacc

    sacc = lax.fori_loop(0, _BPW, row_body, zero)
    racc[...] = sacc

    pltpu.sync_copy(parts, xp_out.at[pl.ds(base, _BPW)])
    pltpu.sync_copy(racc, reg_out.at[wid])


_sc_gather_dot = functools.partial(
    pl.kernel,
    mesh=plsc.VectorSubcoreMesh(core_axis_name="c", subcore_axis_name="s"),
    compiler_params=pltpu.CompilerParams(use_tc_tiling_on_sc=False),
    out_type=[
        jax.ShapeDtypeStruct((_BATCH, _L), jnp.float32),
        jax.ShapeDtypeStruct((_NW, _L), jnp.float32),
    ],
    scratch_types=[
        pltpu.VMEM((_NCHUNK, _CHUNK), jnp.int32),
        pltpu.VMEM((_NCHUNK, _CHUNK), jnp.int32),
        pltpu.VMEM((_NCHUNK, _CHUNK), jnp.int32),
        pltpu.VMEM((_BPW, _D), jnp.float32),
        pltpu.VMEM((_BPW, _D), jnp.float32),
        pltpu.VMEM((_BPW, _D), jnp.float32),
        pltpu.VMEM((_BPW, _L), jnp.float32),
        pltpu.VMEM((_L,), jnp.float32),
        pltpu.SemaphoreType.DMA,
    ],
)(_sc_body)


def _finish_body(xp_ref, regp_ref, loss_ref, reg_ref):
    xp = xp_ref[...]
    grp = lax.broadcasted_iota(jnp.int32, (128, 8), 0) // _L
    col = lax.broadcasted_iota(jnp.int32, (128, 8), 1)
    sel = (grp == col).astype(jnp.float32)
    x = lax.dot_general(xp, sel, (((1,), (0,)), ((), ())),
                        preferred_element_type=jnp.float32)
    ls = jnp.minimum(x, 0.0) - jnp.log1p(jnp.exp(-jnp.abs(x)))
    loss_ref[...] = jnp.reshape(-jnp.sum(ls) * (1.0 / _BATCH), (1, 1))
    reg_ref[...] = jnp.reshape(jnp.sum(regp_ref[...]) * (1.0 / _BATCH), (1, 1))


_finish = pl.pallas_call(
    _finish_body,
    out_shape=(
        jax.ShapeDtypeStruct((1, 1), jnp.float32),
        jax.ShapeDtypeStruct((1, 1), jnp.float32),
    ),
)


def kernel(users, pos, neg, user_table, item_table):
    xp, regp = _sc_gather_dot(users, pos, neg, user_table, item_table)
    loss, reg = _finish(xp.reshape(_BATCH // 8, 128), regp)
    return loss.reshape(()), reg.reshape(())
